# baseline (speedup 1.0000x reference)
import functools

import jax
import jax.numpy as jnp
from jax.experimental import pallas as pl
from jax.experimental.pallas import tpu as pltpu

_VMEM_LIMIT = 48 * 1024 * 1024

_STAGES = (
    (32, 4, 4, True, False, False, ((3, 64), (64, 64))),
    (16, 4, 2, False, True, False, ((64, 128), (128, 128))),
    (8, 8, 1, False, False, False, ((128, 256), (256, 256), (256, 256),
                                    (256, 256))),
    (4, 16, 1, False, False, True, ((256, 512), (512, 512), (512, 512),
                                    (512, 512))),
)
_STAGE_E = ((512, 512), (512, 512), (512, 512), (512, 512))


def _taps(src, w_ref, b, c0, csz, offs):
    acc = None
    for t, off in enumerate(offs):
        y = jnp.dot(src[pl.ds(off + c0, csz), :], w_ref[t],
                    preferred_element_type=jnp.float32)
        acc = y if acc is None else acc + y
    return jnp.maximum(acc + b, 0.0)


def _block_body(*refs, nlayers, R, wp2, pin, h, nb, dense, grp, degroup,
                kwfold):
    x_ref = refs[0]
    wb = refs[1:1 + 2 * nlayers]
    mask_ref = refs[1 + 2 * nlayers]
    o_ref = refs[2 + 2 * nlayers]
    scratch = refs[3 + 2 * nlayers:]
    bufs = scratch[:nlayers - 1]
    res_ref = scratch[nlayers - 1]
    hm_refs = scratch[nlayers:]
    top = wp2 + 1
    offs9 = [kh * wp2 + kw for kh in range(3) for kw in range(3)]
    offs3 = [kh * wp2 for kh in range(3)]

    src = x_ref
    for li in range(nlayers - 1):
        w_ref, b_ref = wb[2 * li], wb[2 * li + 1]
        offs = offs3 if (kwfold and li == 0) else offs9
        dst = bufs[li]
        rows, tco = dst.shape
        chunk = 256 if tco > 256 else 512
        dst[pl.ds(0, top), :] = jnp.zeros((top, tco), dst.dtype)
        tail = rows - top - R
        dst[pl.ds(top + R, tail), :] = jnp.zeros((tail, tco), dst.dtype)
        b = b_ref[...]
        for c0 in range(0, R, chunk):
            csz = min(chunk, R - c0)
            res = _taps(src, w_ref, b, c0, csz, offs)
            res = res * mask_ref[pl.ds(c0, csz), :]
            dst[pl.ds(top + c0, csz), :] = res.astype(dst.dtype)
        src = dst

    w_ref, b_ref = wb[-2], wb[-1]
    offs = offs3 if (kwfold and nlayers == 1) else offs9
    b = b_ref[...]
    cout = res_ref.shape[1]
    chunk = 256 if cout > 256 else 512
    for c0 in range(0, R, chunk):
        csz = min(chunk, R - c0)
        res_ref[pl.ds(c0, csz), :] = _taps(src, w_ref, b, c0, csz, offs)
    rhm = R - wp2 - 1
    lw = cout // grp if degroup else min(cout, 128)
    for g in range(len(hm_refs)):
        lane = pl.ds(g * lw, lw)
        for c0 in range(0, rhm, chunk):
            csz = min(chunk, rhm - c0)
            hm_refs[g][pl.ds(c0, csz), :] = jnp.maximum(
                jnp.maximum(res_ref[pl.ds(c0, csz), lane],
                            res_ref[pl.ds(c0 + 1, csz), lane]),
                jnp.maximum(res_ref[pl.ds(c0 + wp2, csz), lane],
                            res_ref[pl.ds(c0 + wp2 + 1, csz), lane]))
    h2 = w2 = h // 2
    if dense:
        for g in range(len(hm_refs)):
            for py in range(h2):
                for px in range(w2):
                    col = (py * w2 + px) * cout + g * lw
                    o_ref[:, pl.ds(col, lw)] = (
                        hm_refs[g][pl.ds(2 * py * wp2 + 2 * px, nb,
                                         stride=pin), :].astype(o_ref.dtype))
        return
    wo = w2 + 2
    pout = (h2 + 3) * wo
    o_ref[...] = jnp.zeros_like(o_ref)
    for g in range(len(hm_refs)):
        for i in range(nb):
            if degroup:
                lane = pl.ds(0, lw)
                base = (i * grp + g) * pout
            else:
                lane = pl.ds(g * lw, lw)
                base = i * pout
            for py in range(h2):
                row = hm_refs[g][pl.ds(i * pin + 2 * py * wp2, w2, stride=2), :]
                o_ref[pl.ds(base + (py + 1) * wo + 1, w2), lane] = (
                    row.astype(o_ref.dtype))


@functools.lru_cache(maxsize=None)
def _build_block(n_img, h, nb, grp, lcfg, dense, degroup, kwfold):
    nlayers = len(lcfg)
    wp2 = h + 2
    pin = (h + 3) * wp2
    m = h * wp2
    R = (nb - 1) * pin + m
    cout = lcfg[-1][1] * grp
    body = functools.partial(_block_body, nlayers=nlayers, R=R, wp2=wp2,
                             pin=pin, h=h, nb=nb, dense=dense, grp=grp,
                             degroup=degroup, kwfold=kwfold)
    cin0 = lcfg[0][0] * grp * (3 if kwfold else 1)
    in_specs = [pl.BlockSpec((nb * pin, cin0), lambda i: (i, 0))]
    for li, (ci, co) in enumerate(lcfg):
        if kwfold and li == 0:
            in_specs.append(
                pl.BlockSpec((3, 3 * ci * grp, co * grp), lambda i: (0, 0, 0)))
        else:
            in_specs.append(
                pl.BlockSpec((9, ci * grp, co * grp), lambda i: (0, 0, 0)))
        in_specs.append(pl.BlockSpec((1, co * grp), lambda i: (0, 0)))
    in_specs.append(pl.BlockSpec((R, 1), lambda i: (0, 0)))
    h2 = h // 2
    pout = (h2 + 3) * (h2 + 2)
    if dense:
        npix = h2 * h2
        out_shape = jax.ShapeDtypeStruct((n_img, npix * cout), jnp.bfloat16)
        o_spec = pl.BlockSpec((nb, npix * cout), lambda i: (i, 0))
    elif degroup:
        co_img = lcfg[-1][1]
        out_shape = jax.ShapeDtypeStruct((n_img * grp * pout, co_img),
                                         jnp.bfloat16)
        o_spec = pl.BlockSpec((nb * grp * pout, co_img), lambda i: (i, 0))
    else:
        out_shape = jax.ShapeDtypeStruct((n_img * pout, cout), jnp.bfloat16)
        o_spec = pl.BlockSpec((nb * pout, cout), lambda i: (i, 0))
    scratch = [pltpu.VMEM((nb * pin, co * grp), jnp.bfloat16)
               for _, co in lcfg[:-1]]
    scratch.append(pltpu.VMEM((R, cout), jnp.float32))
    lw = cout // grp if degroup else min(cout, 128)
    scratch += [pltpu.VMEM((R - wp2 - 1, lw), jnp.float32)
                for _ in range(cout // lw)]
    return pl.pallas_call(
        body,
        out_shape=out_shape,
        grid=(n_img // nb,),
        in_specs=in_specs,
        out_specs=o_spec,
        scratch_shapes=scratch,
        compiler_params=pltpu.CompilerParams(
            dimension_semantics=("parallel",),
            vmem_limit_bytes=_VMEM_LIMIT),
    )


def _row_mask(R, pin, m, wp2, w):
    r = jnp.arange(R)
    valid = ((r % pin) < m) & ((r % wp2) < w)
    return valid.astype(jnp.float32).reshape(R, 1)


def _block_diag(w9, grp):
    if grp == 1:
        return w9
    t, ci, co = w9.shape
    eye = jnp.eye(grp, dtype=w9.dtype)
    out = jnp.einsum('gh,tio->tgiho', eye, w9)
    return out.reshape(t, grp * ci, grp * co)


def _kwfold_weight(w9, grp):
    t, ci, co = w9.shape
    wk = w9.reshape(3, 3, ci, co)
    eye = jnp.eye(grp, dtype=w9.dtype)
    out = jnp.einsum('gh,kwio->kwgiho', eye, wk)
    return out.reshape(3, 3 * grp * ci, grp * co)


def _stage_e_weight(w9):
    pix = [(0, 0), (0, 1), (1, 0), (1, 1)]
    rows = []
    for yi, xi in pix:
        cols = [w9[(yi - yo + 1) * 3 + (xi - xo + 1)] for yo, xo in pix]
        rows.append(jnp.concatenate(cols, axis=1))
    return jnp.concatenate(rows, axis=0)


def _stage_e_body(x_ref, w0, b0, w1, b1, w2, b2, w3, b3, o_ref):
    y = x_ref[...]
    for w_ref, b_ref in ((w0, b0), (w1, b1), (w2, b2), (w3, b3)):
        d = w_ref.shape[1]
        step = min(512, d)
        parts = []
        for n0 in range(0, d, step):
            acc = None
            for k0 in range(0, d, step):
                z = jnp.dot(y[:, k0:k0 + step], w_ref[pl.ds(k0, step),
                                                      pl.ds(n0, step)],
                            preferred_element_type=jnp.float32)
                acc = z if acc is None else acc + z
            acc = acc + b_ref[:, pl.ds(n0, step)]
            parts.append(jnp.maximum(acc, 0.0).astype(jnp.bfloat16))
        y = parts[0] if len(parts) == 1 else jnp.concatenate(parts, axis=1)
    c = y.shape[1] // 4
    p = jnp.maximum(jnp.maximum(y[:, :c], y[:, c:2 * c]),
                    jnp.maximum(y[:, 2 * c:3 * c], y[:, 3 * c:]))
    o_ref[...] = p


@functools.lru_cache(maxsize=None)
def _build_stage_e(n_img, cin):
    return pl.pallas_call(
        _stage_e_body,
        out_shape=jax.ShapeDtypeStruct((n_img, cin), jnp.bfloat16),
        compiler_params=pltpu.CompilerParams(
            vmem_limit_bytes=_VMEM_LIMIT),
    )


def _fc01_body(x_ref, w0_ref, b0_ref, w1_ref, b1_ref, o_ref, y0_ref):
    d0 = w0_ref.shape[1]
    for n0 in range(0, d0, 512):
        nsz = min(512, d0 - n0)
        acc = jnp.dot(x_ref[...], w0_ref[:, pl.ds(n0, nsz)],
                      preferred_element_type=jnp.float32)
        acc = acc + b0_ref[:, pl.ds(n0, nsz)]
        y0_ref[:, pl.ds(n0, nsz)] = jnp.maximum(acc, 0.0).astype(jnp.bfloat16)
    dn = o_ref.shape[1]
    for n0 in range(0, dn, 512):
        nsz = min(512, dn - n0)
        acc = None
        for k0 in range(0, d0, 1024):
            ksz = min(1024, d0 - k0)
            y = jnp.dot(y0_ref[:, pl.ds(k0, ksz)],
                        w1_ref[pl.ds(k0, ksz), pl.ds(n0, nsz)],
                        preferred_element_type=jnp.float32)
            acc = y if acc is None else acc + y
        acc = acc + b1_ref[:, pl.ds(n0, nsz)]
        o_ref[:, pl.ds(n0, nsz)] = jnp.maximum(acc, 0.0).astype(jnp.bfloat16)


@functools.lru_cache(maxsize=None)
def _build_fc01(bsz, k0, d0, d1):
    tn = d1 // 2
    return pl.pallas_call(
        _fc01_body,
        out_shape=jax.ShapeDtypeStruct((bsz, d1), jnp.bfloat16),
        grid=(2,),
        in_specs=[pl.BlockSpec((bsz, k0), lambda j: (0, 0)),
                  pl.BlockSpec((k0, d0), lambda j: (0, 0)),
                  pl.BlockSpec((1, d0), lambda j: (0, 0)),
                  pl.BlockSpec((d0, tn), lambda j: (0, j)),
                  pl.BlockSpec((1, tn), lambda j: (0, j))],
        out_specs=pl.BlockSpec((bsz, tn), lambda j: (0, j)),
        scratch_shapes=[pltpu.VMEM((bsz, d0), jnp.bfloat16)],
        compiler_params=pltpu.CompilerParams(
            dimension_semantics=("parallel",),
            vmem_limit_bytes=_VMEM_LIMIT),
    )


def _fc2_body(x_ref, w_ref, b_ref, o_ref):
    k = x_ref.shape[1]
    acc = None
    for k0 in range(0, k, 1024):
        ksz = min(1024, k - k0)
        y = jnp.dot(x_ref[:, pl.ds(k0, ksz)], w_ref[pl.ds(k0, ksz), :],
                    preferred_element_type=jnp.float32)
        acc = y if acc is None else acc + y
    o_ref[...] = acc + b_ref[...]


@functools.lru_cache(maxsize=None)
def _build_fc2(bsz, k, n):
    return pl.pallas_call(
        _fc2_body,
        out_shape=jax.ShapeDtypeStruct((bsz, n), jnp.float32),
        grid=(1,),
        in_specs=[pl.BlockSpec((bsz, k), lambda i: (0, 0)),
                  pl.BlockSpec((k, n), lambda i: (0, 0)),
                  pl.BlockSpec((1, n), lambda i: (0, 0))],
        out_specs=pl.BlockSpec((bsz, n), lambda i: (0, 0)),
        compiler_params=pltpu.CompilerParams(
            vmem_limit_bytes=_VMEM_LIMIT),
    )


def _forward(x_nchw, conv_params, fc_params):
    n, cin, h, w = x_nchw.shape
    g0 = _STAGES[0][2]
    x = jnp.transpose(x_nchw, (0, 2, 3, 1)).astype(jnp.bfloat16)
    x = jnp.pad(x, ((0, 0), (1, 2), (1, 1), (0, 0)))
    pin0 = (h + 3) * (w + 2)
    x = x.reshape(n // g0, g0, pin0, cin)
    x = jnp.transpose(x, (0, 2, 1, 3)).reshape((n // g0) * pin0, g0 * cin)
    x = jnp.concatenate(
        [x, jnp.pad(x[1:], ((0, 1), (0, 0))),
         jnp.pad(x[2:], ((0, 2), (0, 0)))], axis=1)

    li = 0
    prev_grp = g0
    for hh, nb, grp, kwfold, degroup, dense, lcfg in _STAGES:
        if grp != prev_grp:
            rows, ch = x.shape
            nimg_p = n // prev_grp
            x = x.reshape(nimg_p, rows // nimg_p, prev_grp // grp,
                          ch // (prev_grp // grp))
            x = jnp.transpose(x, (0, 2, 1, 3))
            x = x.reshape((n // grp) * (rows // nimg_p),
                          ch // (prev_grp // grp))
        prev_grp = 1 if (degroup or dense) else grp
        wp2 = hh + 2
        pin = (hh + 3) * wp2
        m = hh * wp2
        R = (nb - 1) * pin + m
        mask = _row_mask(R, pin, m, wp2, hh)
        args = [x]
        for lidx in range(len(lcfg)):
            w9, b = conv_params[li]
            if kwfold and lidx == 0:
                args += [_kwfold_weight(w9, grp), jnp.tile(b, (1, grp))]
            else:
                args += [_block_diag(w9, grp), jnp.tile(b, (1, grp))]
            li += 1
        args.append(mask)
        x = _build_block(n // grp, hh, nb, grp, lcfg, dense, degroup,
                         kwfold)(*args)

    eargs = [x]
    for _ in _STAGE_E:
        w9, b = conv_params[li]
        eargs += [_stage_e_weight(w9), jnp.tile(b, (1, 4))]
        li += 1
    x = _build_stage_e(n, _STAGE_E[-1][1])(*eargs)

    (w0, b0), (w1, b1), (w2, b2) = fc_params
    y = _build_fc01(n, w0.shape[0], w0.shape[1], w1.shape[1])(
        x, w0, b0, w1, b1)
    y = _build_fc2(n, w1.shape[1], w2.shape[1])(y, w2, b2)
    return y[:, :50]


_forward_jit = jax.jit(_forward)


def kernel(x, conv_w_0, conv_b_0, conv_w_1, conv_b_1, conv_w_2, conv_b_2,
           conv_w_3, conv_b_3, conv_w_4, conv_b_4, conv_w_5, conv_b_5,
           conv_w_6, conv_b_6, conv_w_7, conv_b_7, conv_w_8, conv_b_8,
           conv_w_9, conv_b_9, conv_w_10, conv_b_10, conv_w_11, conv_b_11,
           conv_w_12, conv_b_12, conv_w_13, conv_b_13, conv_w_14, conv_b_14,
           conv_w_15, conv_b_15, fc_w0, fc_b0, fc_w1, fc_b1, fc_w2, fc_b2):
    conv_params = [
        (conv_w_0, conv_b_0), (conv_w_1, conv_b_1),
        (conv_w_2, conv_b_2), (conv_w_3, conv_b_3),
        (conv_w_4, conv_b_4), (conv_w_5, conv_b_5),
        (conv_w_6, conv_b_6), (conv_w_7, conv_b_7),
        (conv_w_8, conv_b_8), (conv_w_9, conv_b_9),
        (conv_w_10, conv_b_10), (conv_w_11, conv_b_11),
        (conv_w_12, conv_b_12), (conv_w_13, conv_b_13),
        (conv_w_14, conv_b_14), (conv_w_15, conv_b_15),
    ]
    fc_params = [(fc_w0, fc_b0), (fc_w1, fc_b1), (fc_w2, fc_b2)]
    return _forward_jit(x, conv_params, fc_params)


# speedup vs baseline: 1.9720x; 1.9720x over previous
import functools

import jax
import jax.numpy as jnp
from jax.experimental import pallas as pl
from jax.experimental.pallas import tpu as pltpu

_VMEM_LIMIT = 48 * 1024 * 1024

_STAGES = (
    (32, 4, 4, True, False, False, ((3, 64), (64, 64))),
    (16, 4, 2, False, True, False, ((64, 128), (128, 128))),
    (8, 8, 1, False, False, False, ((128, 256), (256, 256), (256, 256),
                                    (256, 256))),
    (4, 16, 1, False, False, True, ((256, 512), (512, 512), (512, 512),
                                    (512, 512))),
)
_STAGE_E = ((512, 512), (512, 512), (512, 512), (512, 512))


def _taps(src, w_ref, b, c0, csz, offs):
    acc = None
    for t, off in enumerate(offs):
        y = jnp.dot(src[pl.ds(off + c0, csz), :], w_ref[t],
                    preferred_element_type=jnp.float32)
        acc = y if acc is None else acc + y
    return jnp.maximum(acc + b, 0.0)


def _block_body(*refs, nlayers, R, wp2, pin, h, nb, dense, grp, degroup,
                kwfold):
    x_ref = refs[0]
    wb = refs[1:1 + 2 * nlayers]
    mask_ref = refs[1 + 2 * nlayers]
    o_ref = refs[2 + 2 * nlayers]
    scratch = refs[3 + 2 * nlayers:]
    bufs = scratch[:nlayers - 1]
    res_ref = scratch[nlayers - 1]
    hm_refs = scratch[nlayers:]
    top = wp2 + 1
    offs9 = [kh * wp2 + kw for kh in range(3) for kw in range(3)]
    offs3 = [kh * wp2 for kh in range(3)]

    src = x_ref
    for li in range(nlayers - 1):
        w_ref, b_ref = wb[2 * li], wb[2 * li + 1]
        offs = offs3 if (kwfold and li == 0) else offs9
        dst = bufs[li]
        rows, tco = dst.shape
        chunk = 256 if tco > 256 else 512
        dst[pl.ds(0, top), :] = jnp.zeros((top, tco), dst.dtype)
        tail = rows - top - R
        dst[pl.ds(top + R, tail), :] = jnp.zeros((tail, tco), dst.dtype)
        b = b_ref[...]
        for c0 in range(0, R, chunk):
            csz = min(chunk, R - c0)
            res = _taps(src, w_ref, b, c0, csz, offs)
            res = res * mask_ref[pl.ds(c0, csz), :]
            dst[pl.ds(top + c0, csz), :] = res.astype(dst.dtype)
        src = dst

    w_ref, b_ref = wb[-2], wb[-1]
    offs = offs3 if (kwfold and nlayers == 1) else offs9
    b = b_ref[...]
    cout = res_ref.shape[1]
    chunk = 256 if cout > 256 else 512
    for c0 in range(0, R, chunk):
        csz = min(chunk, R - c0)
        res_ref[pl.ds(c0, csz), :] = _taps(src, w_ref, b, c0, csz, offs)
    rhm = R - wp2 - 1
    lw = cout // grp if degroup else min(cout, 128)
    for g in range(len(hm_refs)):
        lane = pl.ds(g * lw, lw)
        for c0 in range(0, rhm, chunk):
            csz = min(chunk, rhm - c0)
            hm_refs[g][pl.ds(c0, csz), :] = jnp.maximum(
                jnp.maximum(res_ref[pl.ds(c0, csz), lane],
                            res_ref[pl.ds(c0 + 1, csz), lane]),
                jnp.maximum(res_ref[pl.ds(c0 + wp2, csz), lane],
                            res_ref[pl.ds(c0 + wp2 + 1, csz), lane]))
    h2 = w2 = h // 2
    if dense:
        for g in range(len(hm_refs)):
            for py in range(h2):
                for px in range(w2):
                    col = (py * w2 + px) * cout + g * lw
                    o_ref[:, pl.ds(col, lw)] = (
                        hm_refs[g][pl.ds(2 * py * wp2 + 2 * px, nb,
                                         stride=pin), :].astype(o_ref.dtype))
        return
    wo = w2 + 2
    pout = (h2 + 3) * wo
    o_ref[...] = jnp.zeros_like(o_ref)
    for g in range(len(hm_refs)):
        for i in range(nb):
            if degroup:
                lane = pl.ds(0, lw)
                base = (i * grp + g) * pout
            else:
                lane = pl.ds(g * lw, lw)
                base = i * pout
            for py in range(h2):
                row = hm_refs[g][pl.ds(i * pin + 2 * py * wp2, w2, stride=2), :]
                o_ref[pl.ds(base + (py + 1) * wo + 1, w2), lane] = (
                    row.astype(o_ref.dtype))


@functools.lru_cache(maxsize=None)
def _build_block(n_img, h, nb, grp, lcfg, dense, degroup, kwfold):
    nlayers = len(lcfg)
    wp2 = h + 2
    pin = (h + 3) * wp2
    m = h * wp2
    R = (nb - 1) * pin + m
    cout = lcfg[-1][1] * grp
    body = functools.partial(_block_body, nlayers=nlayers, R=R, wp2=wp2,
                             pin=pin, h=h, nb=nb, dense=dense, grp=grp,
                             degroup=degroup, kwfold=kwfold)
    cin0 = lcfg[0][0] * grp * (3 if kwfold else 1)
    in_specs = [pl.BlockSpec((nb * pin, cin0), lambda i: (i, 0))]
    for li, (ci, co) in enumerate(lcfg):
        if kwfold and li == 0:
            in_specs.append(
                pl.BlockSpec((3, 3 * ci * grp, co * grp), lambda i: (0, 0, 0)))
        else:
            in_specs.append(
                pl.BlockSpec((9, ci * grp, co * grp), lambda i: (0, 0, 0)))
        in_specs.append(pl.BlockSpec((1, co * grp), lambda i: (0, 0)))
    in_specs.append(pl.BlockSpec((R, 1), lambda i: (0, 0)))
    h2 = h // 2
    pout = (h2 + 3) * (h2 + 2)
    if dense:
        npix = h2 * h2
        out_shape = jax.ShapeDtypeStruct((n_img, npix * cout), jnp.bfloat16)
        o_spec = pl.BlockSpec((nb, npix * cout), lambda i: (i, 0))
    elif degroup:
        co_img = lcfg[-1][1]
        out_shape = jax.ShapeDtypeStruct((n_img * grp * pout, co_img),
                                         jnp.bfloat16)
        o_spec = pl.BlockSpec((nb * grp * pout, co_img), lambda i: (i, 0))
    else:
        out_shape = jax.ShapeDtypeStruct((n_img * pout, cout), jnp.bfloat16)
        o_spec = pl.BlockSpec((nb * pout, cout), lambda i: (i, 0))
    scratch = [pltpu.VMEM((nb * pin, co * grp), jnp.bfloat16)
               for _, co in lcfg[:-1]]
    scratch.append(pltpu.VMEM((R, cout), jnp.float32))
    lw = cout // grp if degroup else min(cout, 128)
    scratch += [pltpu.VMEM((R - wp2 - 1, lw), jnp.float32)
                for _ in range(cout // lw)]
    return pl.pallas_call(
        body,
        out_shape=out_shape,
        grid=(n_img // nb,),
        in_specs=in_specs,
        out_specs=o_spec,
        scratch_shapes=scratch,
        compiler_params=pltpu.CompilerParams(
            dimension_semantics=("parallel",),
            vmem_limit_bytes=_VMEM_LIMIT),
    )


def _row_mask(R, pin, m, wp2, w):
    r = jnp.arange(R)
    valid = ((r % pin) < m) & ((r % wp2) < w)
    return valid.astype(jnp.float32).reshape(R, 1)


def _block_diag(w9, grp):
    if grp == 1:
        return w9
    t, ci, co = w9.shape
    eye = jnp.eye(grp, dtype=w9.dtype)
    out = jnp.einsum('gh,tio->tgiho', eye, w9)
    return out.reshape(t, grp * ci, grp * co)


def _kwfold_weight(w9, grp):
    t, ci, co = w9.shape
    wk = w9.reshape(3, 3, ci, co)
    eye = jnp.eye(grp, dtype=w9.dtype)
    out = jnp.einsum('gh,kwio->kwgiho', eye, wk)
    return out.reshape(3, 3 * grp * ci, grp * co)


def _stage_e_body(x_ref, w0, b0, w1, b1, w2, b2, w3, b3, o_ref):
    pix = ((0, 0), (0, 1), (1, 0), (1, 1))
    y = x_ref[...]
    c = y.shape[1] // 4
    for w_ref, b_ref in ((w0, b0), (w1, b1), (w2, b2), (w3, b3)):
        b = b_ref[...]
        parts = []
        for yo, xo in pix:
            acc = None
            for pi, (yi, xi) in enumerate(pix):
                t = (yi - yo + 1) * 3 + (xi - xo + 1)
                z = jnp.dot(y[:, pi * c:(pi + 1) * c], w_ref[t],
                            preferred_element_type=jnp.float32)
                acc = z if acc is None else acc + z
            parts.append(jnp.maximum(acc + b, 0.0).astype(jnp.bfloat16))
        y = jnp.concatenate(parts, axis=1)
    p = jnp.maximum(jnp.maximum(y[:, :c], y[:, c:2 * c]),
                    jnp.maximum(y[:, 2 * c:3 * c], y[:, 3 * c:]))
    o_ref[...] = p


@functools.lru_cache(maxsize=None)
def _build_stage_e(n_img, cin):
    return pl.pallas_call(
        _stage_e_body,
        out_shape=jax.ShapeDtypeStruct((n_img, cin), jnp.bfloat16),
        compiler_params=pltpu.CompilerParams(
            vmem_limit_bytes=_VMEM_LIMIT),
    )


def _fc01_body(x_ref, w0_ref, b0_ref, w1_ref, b1_ref, o_ref, y0_ref):
    d0 = w0_ref.shape[1]
    for n0 in range(0, d0, 512):
        nsz = min(512, d0 - n0)
        acc = jnp.dot(x_ref[...], w0_ref[:, pl.ds(n0, nsz)],
                      preferred_element_type=jnp.float32)
        acc = acc + b0_ref[:, pl.ds(n0, nsz)]
        y0_ref[:, pl.ds(n0, nsz)] = jnp.maximum(acc, 0.0).astype(jnp.bfloat16)
    dn = o_ref.shape[1]
    for n0 in range(0, dn, 512):
        nsz = min(512, dn - n0)
        acc = None
        for k0 in range(0, d0, 1024):
            ksz = min(1024, d0 - k0)
            y = jnp.dot(y0_ref[:, pl.ds(k0, ksz)],
                        w1_ref[pl.ds(k0, ksz), pl.ds(n0, nsz)],
                        preferred_element_type=jnp.float32)
            acc = y if acc is None else acc + y
        acc = acc + b1_ref[:, pl.ds(n0, nsz)]
        o_ref[:, pl.ds(n0, nsz)] = jnp.maximum(acc, 0.0).astype(jnp.bfloat16)


@functools.lru_cache(maxsize=None)
def _build_fc01(bsz, k0, d0, d1):
    tn = d1 // 2
    return pl.pallas_call(
        _fc01_body,
        out_shape=jax.ShapeDtypeStruct((bsz, d1), jnp.bfloat16),
        grid=(2,),
        in_specs=[pl.BlockSpec((bsz, k0), lambda j: (0, 0)),
                  pl.BlockSpec((k0, d0), lambda j: (0, 0)),
                  pl.BlockSpec((1, d0), lambda j: (0, 0)),
                  pl.BlockSpec((d0, tn), lambda j: (0, j)),
                  pl.BlockSpec((1, tn), lambda j: (0, j))],
        out_specs=pl.BlockSpec((bsz, tn), lambda j: (0, j)),
        scratch_shapes=[pltpu.VMEM((bsz, d0), jnp.bfloat16)],
        compiler_params=pltpu.CompilerParams(
            dimension_semantics=("parallel",),
            vmem_limit_bytes=_VMEM_LIMIT),
    )


def _fc2_body(x_ref, w_ref, b_ref, o_ref):
    k = x_ref.shape[1]
    acc = None
    for k0 in range(0, k, 1024):
        ksz = min(1024, k - k0)
        y = jnp.dot(x_ref[:, pl.ds(k0, ksz)], w_ref[pl.ds(k0, ksz), :],
                    preferred_element_type=jnp.float32)
        acc = y if acc is None else acc + y
    o_ref[...] = acc + b_ref[...]


@functools.lru_cache(maxsize=None)
def _build_fc2(bsz, k, n):
    return pl.pallas_call(
        _fc2_body,
        out_shape=jax.ShapeDtypeStruct((bsz, n), jnp.float32),
        grid=(1,),
        in_specs=[pl.BlockSpec((bsz, k), lambda i: (0, 0)),
                  pl.BlockSpec((k, n), lambda i: (0, 0)),
                  pl.BlockSpec((1, n), lambda i: (0, 0))],
        out_specs=pl.BlockSpec((bsz, n), lambda i: (0, 0)),
        compiler_params=pltpu.CompilerParams(
            vmem_limit_bytes=_VMEM_LIMIT),
    )


def _forward(x_nchw, conv_params, fc_params):
    n, cin, h, w = x_nchw.shape
    g0 = _STAGES[0][2]
    x = jnp.transpose(x_nchw, (0, 2, 3, 1)).astype(jnp.bfloat16)
    x = jnp.pad(x, ((0, 0), (1, 2), (1, 1), (0, 0)))
    pin0 = (h + 3) * (w + 2)
    x = x.reshape(n // g0, g0, pin0, cin)
    x = jnp.transpose(x, (0, 2, 1, 3)).reshape((n // g0) * pin0, g0 * cin)
    x = jnp.concatenate(
        [x, jnp.pad(x[1:], ((0, 1), (0, 0))),
         jnp.pad(x[2:], ((0, 2), (0, 0)))], axis=1)

    li = 0
    prev_grp = g0
    for hh, nb, grp, kwfold, degroup, dense, lcfg in _STAGES:
        if grp != prev_grp:
            rows, ch = x.shape
            nimg_p = n // prev_grp
            x = x.reshape(nimg_p, rows // nimg_p, prev_grp // grp,
                          ch // (prev_grp // grp))
            x = jnp.transpose(x, (0, 2, 1, 3))
            x = x.reshape((n // grp) * (rows // nimg_p),
                          ch // (prev_grp // grp))
        prev_grp = 1 if (degroup or dense) else grp
        wp2 = hh + 2
        pin = (hh + 3) * wp2
        m = hh * wp2
        R = (nb - 1) * pin + m
        mask = _row_mask(R, pin, m, wp2, hh)
        args = [x]
        for lidx in range(len(lcfg)):
            w9, b = conv_params[li]
            if kwfold and lidx == 0:
                args += [_kwfold_weight(w9, grp), jnp.tile(b, (1, grp))]
            else:
                args += [_block_diag(w9, grp), jnp.tile(b, (1, grp))]
            li += 1
        args.append(mask)
        x = _build_block(n // grp, hh, nb, grp, lcfg, dense, degroup,
                         kwfold)(*args)

    eargs = [x]
    for _ in _STAGE_E:
        w9, b = conv_params[li]
        eargs += [w9, b]
        li += 1
    x = _build_stage_e(n, _STAGE_E[-1][1])(*eargs)

    (w0, b0), (w1, b1), (w2, b2) = fc_params
    y = _build_fc01(n, w0.shape[0], w0.shape[1], w1.shape[1])(
        x, w0, b0, w1, b1)
    y = _build_fc2(n, w1.shape[1], w2.shape[1])(y, w2, b2)
    return y[:, :50]


_forward_jit = jax.jit(_forward)


def kernel(x, conv_w_0, conv_b_0, conv_w_1, conv_b_1, conv_w_2, conv_b_2,
           conv_w_3, conv_b_3, conv_w_4, conv_b_4, conv_w_5, conv_b_5,
           conv_w_6, conv_b_6, conv_w_7, conv_b_7, conv_w_8, conv_b_8,
           conv_w_9, conv_b_9, conv_w_10, conv_b_10, conv_w_11, conv_b_11,
           conv_w_12, conv_b_12, conv_w_13, conv_b_13, conv_w_14, conv_b_14,
           conv_w_15, conv_b_15, fc_w0, fc_b0, fc_w1, fc_b1, fc_w2, fc_b2):
    conv_params = [
        (conv_w_0, conv_b_0), (conv_w_1, conv_b_1),
        (conv_w_2, conv_b_2), (conv_w_3, conv_b_3),
        (conv_w_4, conv_b_4), (conv_w_5, conv_b_5),
        (conv_w_6, conv_b_6), (conv_w_7, conv_b_7),
        (conv_w_8, conv_b_8), (conv_w_9, conv_b_9),
        (conv_w_10, conv_b_10), (conv_w_11, conv_b_11),
        (conv_w_12, conv_b_12), (conv_w_13, conv_b_13),
        (conv_w_14, conv_b_14), (conv_w_15, conv_b_15),
    ]
    fc_params = [(fc_w0, fc_b0), (fc_w1, fc_b1), (fc_w2, fc_b2)]
    return _forward_jit(x, conv_params, fc_params)
